# baseline (speedup 1.0000x reference)
import functools

import jax
import jax.numpy as jnp
from jax import lax
from jax.experimental import pallas as pl
from jax.experimental.pallas import tpu as pltpu
from jax.experimental.pallas import tpu_sc as plsc

_PROBABILITY = 0.12
_T = 2048
_ROWS = 8192
_NW = 32
_RPW = _ROWS // _NW
_NR = 16
_NBLK = _RPW // _NR
_L = 16


def _src_indices():
    key = jax.random.key(42)
    k_replace, k_dir = jax.random.split(key)
    replace = jax.random.uniform(k_replace, (_T,)) < _PROBABILITY
    direction = jnp.where(jax.random.uniform(k_dir, (_T,)) < 0.5, -1, 1)
    pos = jnp.arange(_T)
    neighbor = jnp.where(
        pos == 0, 1, jnp.where(pos == _T - 1, _T - 2, pos + direction)
    )
    return jnp.where(replace, neighbor, pos).astype(jnp.int32)


@functools.partial(
    pl.kernel,
    mesh=plsc.VectorSubcoreMesh(core_axis_name="c", subcore_axis_name="s"),
    out_type=jax.ShapeDtypeStruct((_ROWS * _T,), jnp.float32),
    scratch_types=[
        pltpu.VMEM((_T,), jnp.int32),
        pltpu.VMEM((_NR * _T,), jnp.float32),
        pltpu.VMEM((_NR * _T,), jnp.float32),
    ],
    compiler_params=pltpu.CompilerParams(needs_layout_passes=False),
)
def _sc_jitter(x_hbm, src_hbm, out_hbm, src_v, in_v, out_v):
    wid = lax.axis_index("s") * 2 + lax.axis_index("c")
    pltpu.sync_copy(src_hbm, src_v)

    def blk_body(b, carry):
        base = (wid * _RPW + b * _NR) * _T
        pltpu.sync_copy(x_hbm.at[pl.ds(base, _NR * _T)], in_v)

        def row_body(r, carry2):
            roff = r * _T

            def chunk_body(j, carry3):
                off = j * _L
                col = src_v[pl.ds(off, _L)]
                g = plsc.load_gather(in_v, [col + roff])
                out_v[pl.ds(roff + off, _L)] = g
                return carry3

            lax.fori_loop(0, _T // _L, chunk_body, 0, unroll=8)
            return carry2

        lax.fori_loop(0, _NR, row_body, 0)
        pltpu.sync_copy(out_v, out_hbm.at[pl.ds(base, _NR * _T)])
        return carry

    lax.fori_loop(0, _NBLK, blk_body, 0)


def kernel(quantized):
    B, C, T = quantized.shape
    x1d = quantized.reshape(-1)
    src = _src_indices()
    out = _sc_jitter(x1d, src)
    return out.reshape(B, C, T)


# speedup vs baseline: 1.6831x; 1.6831x over previous
import functools

import jax
import jax.numpy as jnp
from jax import lax
from jax.experimental import pallas as pl
from jax.experimental.pallas import tpu as pltpu
from jax.experimental.pallas import tpu_sc as plsc

_PROBABILITY = 0.12
_T = 2048
_ROWS = 8192
_NW = 32
_RPW = _ROWS // _NW
_NR = 8
_NBLK = _RPW // _NR
_L = 16
_BLK = _NR * _T


def _src_indices():
    key = jax.random.key(42)
    k_replace, k_dir = jax.random.split(key)
    replace = jax.random.uniform(k_replace, (_T,)) < _PROBABILITY
    direction = jnp.where(jax.random.uniform(k_dir, (_T,)) < 0.5, -1, 1)
    pos = jnp.arange(_T)
    neighbor = jnp.where(
        pos == 0, 1, jnp.where(pos == _T - 1, _T - 2, pos + direction)
    )
    return jnp.where(replace, neighbor, pos).astype(jnp.int32)


@functools.partial(
    pl.kernel,
    mesh=plsc.VectorSubcoreMesh(core_axis_name="c", subcore_axis_name="s"),
    out_type=jax.ShapeDtypeStruct((_ROWS * _T,), jnp.float32),
    scratch_types=[
        pltpu.VMEM((_T,), jnp.int32),
        pltpu.VMEM((_BLK,), jnp.float32),
        pltpu.VMEM((_BLK,), jnp.float32),
        pltpu.VMEM((_BLK,), jnp.float32),
        pltpu.VMEM((_BLK,), jnp.float32),
        pltpu.SemaphoreType.DMA,
        pltpu.SemaphoreType.DMA,
        pltpu.SemaphoreType.DMA,
        pltpu.SemaphoreType.DMA,
    ],
    compiler_params=pltpu.CompilerParams(needs_layout_passes=False),
)
def _sc_jitter(x_hbm, src_hbm, out_hbm, src_v, in0, in1, out0, out1,
               sem_i0, sem_i1, sem_o0, sem_o1):
    wid = lax.axis_index("s") * 2 + lax.axis_index("c")
    row0 = wid * _RPW
    pltpu.sync_copy(src_hbm, src_v)

    ins = (in0, in1)
    outs = (out0, out1)
    sems_i = (sem_i0, sem_i1)
    sems_o = (sem_o0, sem_o1)

    def in_slice(g):
        return x_hbm.at[pl.ds((row0 + g * _NR) * _T, _BLK)]

    def out_slice(g):
        return out_hbm.at[pl.ds((row0 + g * _NR) * _T, _BLK)]

    pltpu.make_async_copy(in_slice(0), ins[0], sems_i[0]).start()

    def process(g, ph):
        pltpu.make_async_copy(in_slice(g), ins[ph], sems_i[ph]).wait()
        @pl.when(g + 1 < _NBLK)
        def _():
            pltpu.make_async_copy(
                in_slice(g + 1), ins[1 - ph], sems_i[1 - ph]
            ).start()

        @pl.when(g >= 2)
        def _():
            pltpu.make_async_copy(outs[ph], out_slice(g - 2), sems_o[ph]).wait()

        def chunk(j, carry):
            off = j * _L
            col = src_v[pl.ds(off, _L)]
            for r in range(_NR):
                gat = plsc.load_gather(ins[ph], [col + r * _T])
                outs[ph][pl.ds(r * _T + off, _L)] = gat
            return carry

        lax.fori_loop(0, _T // _L, chunk, 0)
        pltpu.make_async_copy(outs[ph], out_slice(g), sems_o[ph]).start()

    def pair(i, carry):
        process(i * 2, 0)
        process(i * 2 + 1, 1)
        return carry

    lax.fori_loop(0, _NBLK // 2, pair, 0)
    pltpu.make_async_copy(outs[0], out_slice(_NBLK - 2), sems_o[0]).wait()
    pltpu.make_async_copy(outs[1], out_slice(_NBLK - 1), sems_o[1]).wait()


def kernel(quantized):
    B, C, T = quantized.shape
    x1d = quantized.reshape(-1)
    src = _src_indices()
    out = _sc_jitter(x1d, src)
    return out.reshape(B, C, T)


# speedup vs baseline: 2.5536x; 1.5172x over previous
import functools

import jax
import jax.numpy as jnp
from jax import lax
from jax.experimental import pallas as pl
from jax.experimental.pallas import tpu as pltpu
from jax.experimental.pallas import tpu_sc as plsc

_PROBABILITY = 0.12
_T = 2048
_ROWS = 8192
_NW = 32
_RPW = _ROWS // _NW
_NR = 8
_NBLK = _RPW // _NR
_L = 16
_BLK = _NR * _T


def _src_indices():
    key = jax.random.key(42)
    k_replace, k_dir = jax.random.split(key)
    replace = jax.random.uniform(k_replace, (_T,)) < _PROBABILITY
    direction = jnp.where(jax.random.uniform(k_dir, (_T,)) < 0.5, -1, 1)
    pos = jnp.arange(_T)
    neighbor = jnp.where(
        pos == 0, 1, jnp.where(pos == _T - 1, _T - 2, pos + direction)
    )
    return jnp.where(replace, neighbor, pos).astype(jnp.int32)


@functools.partial(
    pl.kernel,
    mesh=plsc.VectorSubcoreMesh(core_axis_name="c", subcore_axis_name="s"),
    out_type=jax.ShapeDtypeStruct((_ROWS * _T,), jnp.float32),
    scratch_types=[
        pltpu.VMEM((_T,), jnp.int32),
        pltpu.VMEM((_BLK,), jnp.float32),
        pltpu.VMEM((_BLK,), jnp.float32),
        pltpu.VMEM((_BLK,), jnp.float32),
        pltpu.VMEM((_BLK,), jnp.float32),
        pltpu.SemaphoreType.DMA,
        pltpu.SemaphoreType.DMA,
        pltpu.SemaphoreType.DMA,
        pltpu.SemaphoreType.DMA,
    ],
    compiler_params=pltpu.CompilerParams(needs_layout_passes=False),
)
def _sc_jitter(x_hbm, src_hbm, out_hbm, src_v, in0, in1, out0, out1,
               sem_i0, sem_i1, sem_o0, sem_o1):
    wid = lax.axis_index("s") * 2 + lax.axis_index("c")
    row0 = wid * _RPW
    pltpu.sync_copy(src_hbm, src_v)

    ins = (in0, in1)
    outs = (out0, out1)
    sems_i = (sem_i0, sem_i1)
    sems_o = (sem_o0, sem_o1)

    def in_slice(g):
        return x_hbm.at[pl.ds((row0 + g * _NR) * _T, _BLK)]

    def out_slice(g):
        return out_hbm.at[pl.ds((row0 + g * _NR) * _T, _BLK)]

    pltpu.make_async_copy(in_slice(0), ins[0], sems_i[0]).start()

    def process(g, ph):
        pltpu.make_async_copy(in_slice(g), ins[ph], sems_i[ph]).wait()
        @pl.when(g + 1 < _NBLK)
        def _():
            pltpu.make_async_copy(
                in_slice(g + 1), ins[1 - ph], sems_i[1 - ph]
            ).start()

        @pl.when(g >= 2)
        def _():
            pltpu.make_async_copy(outs[ph], out_slice(g - 2), sems_o[ph]).wait()

        @plsc.parallel_loop(0, _T // _L, unroll=4)
        def chunk(j):
            off = j * _L
            col = src_v[pl.ds(off, _L)]
            for r in range(_NR):
                gat = plsc.load_gather(ins[ph], [col + r * _T])
                outs[ph][pl.ds(r * _T + off, _L)] = gat
        pltpu.make_async_copy(outs[ph], out_slice(g), sems_o[ph]).start()

    def pair(i, carry):
        process(i * 2, 0)
        process(i * 2 + 1, 1)
        return carry

    lax.fori_loop(0, _NBLK // 2, pair, 0)
    pltpu.make_async_copy(outs[0], out_slice(_NBLK - 2), sems_o[0]).wait()
    pltpu.make_async_copy(outs[1], out_slice(_NBLK - 1), sems_o[1]).wait()


def kernel(quantized):
    B, C, T = quantized.shape
    x1d = quantized.reshape(-1)
    src = _src_indices()
    out = _sc_jitter(x1d, src)
    return out.reshape(B, C, T)


# speedup vs baseline: 6.2044x; 2.4297x over previous
import functools

import jax
import jax.numpy as jnp
from jax import lax
from jax.experimental import pallas as pl
from jax.experimental.pallas import tpu as pltpu
from jax.experimental.pallas import tpu_sc as plsc

_PROBABILITY = 0.12
_T = 2048
_ROWS = 8192
_NW = 32
_RPW = _ROWS // _NW
_NR = 8
_NBLK = _RPW // _NR
_L = 16


def _src_indices():
    key = jax.random.key(42)
    k_replace, k_dir = jax.random.split(key)
    replace = jax.random.uniform(k_replace, (_T,)) < _PROBABILITY
    direction = jnp.where(jax.random.uniform(k_dir, (_T,)) < 0.5, -1, 1)
    pos = jnp.arange(_T)
    neighbor = jnp.where(
        pos == 0, 1, jnp.where(pos == _T - 1, _T - 2, pos + direction)
    )
    return jnp.where(replace, neighbor, pos).astype(jnp.int32)


@functools.partial(
    pl.kernel,
    mesh=plsc.VectorSubcoreMesh(core_axis_name="c", subcore_axis_name="s"),
    out_type=jax.ShapeDtypeStruct((_ROWS, _T), jnp.float32),
    scratch_types=[
        pltpu.VMEM((_T,), jnp.int32),
        pltpu.VMEM((_NR, _T), jnp.float32),
        pltpu.VMEM((_NR, _T), jnp.float32),
        pltpu.VMEM((_NR, _T), jnp.float32),
        pltpu.VMEM((_NR, _T), jnp.float32),
        pltpu.SemaphoreType.DMA,
        pltpu.SemaphoreType.DMA,
        pltpu.SemaphoreType.DMA,
        pltpu.SemaphoreType.DMA,
    ],
    compiler_params=pltpu.CompilerParams(needs_layout_passes=False),
)
def _sc_jitter(x_hbm, src_hbm, out_hbm, src_v,
               in0, in1, out0, out1, sem_i0, sem_i1, sem_o0, sem_o1):
    wid = lax.axis_index("s") * 2 + lax.axis_index("c")
    row0 = wid * _RPW
    pltpu.sync_copy(src_hbm, src_v)

    ins = (in0, in1)
    outs = (out0, out1)
    sems_i = (sem_i0, sem_i1)
    sems_o = (sem_o0, sem_o1)

    def in_slice(g):
        return x_hbm.at[pl.ds(row0 + g * _NR, _NR), :]

    def out_slice(g):
        return out_hbm.at[pl.ds(row0 + g * _NR, _NR), :]

    pltpu.make_async_copy(in_slice(0), ins[0], sems_i[0]).start()

    def process(g, ph):
        pltpu.make_async_copy(in_slice(g), ins[ph], sems_i[ph]).wait()
        @pl.when(g + 1 < _NBLK)
        def _():
            pltpu.make_async_copy(
                in_slice(g + 1), ins[1 - ph], sems_i[1 - ph]
            ).start()

        @pl.when(g >= 2)
        def _():
            pltpu.make_async_copy(outs[ph], out_slice(g - 2), sems_o[ph]).wait()

        @plsc.parallel_loop(0, _T // _L, unroll=4)
        def chunk(j):
            off = j * _L
            col = src_v[pl.ds(off, _L)]
            for r in range(_NR):
                rvec = jnp.full((_L,), r, dtype=jnp.int32)
                gat = plsc.load_gather(ins[ph], [rvec, col])
                outs[ph][r, pl.ds(off, _L)] = gat

        pltpu.make_async_copy(outs[ph], out_slice(g), sems_o[ph]).start()

    def pair(i, carry):
        process(i * 2, 0)
        process(i * 2 + 1, 1)
        return carry

    lax.fori_loop(0, _NBLK // 2, pair, 0)
    pltpu.make_async_copy(outs[0], out_slice(_NBLK - 2), sems_o[0]).wait()
    pltpu.make_async_copy(outs[1], out_slice(_NBLK - 1), sems_o[1]).wait()


def kernel(quantized):
    B, C, T = quantized.shape
    x2d = quantized.reshape(B * C, T)
    src = _src_indices()
    out = _sc_jitter(x2d, src)
    return out.reshape(B, C, T)


# speedup vs baseline: 6.2106x; 1.0010x over previous
import functools

import jax
import jax.numpy as jnp
from jax import lax
from jax.experimental import pallas as pl
from jax.experimental.pallas import tpu as pltpu
from jax.experimental.pallas import tpu_sc as plsc

_PROBABILITY = 0.12
_T = 2048
_ROWS = 8192
_NW = 32
_RPW = _ROWS // _NW
_NR = 8
_NBLK = _RPW // _NR
_L = 16


def _src_indices():
    key = jax.random.key(42)
    k_replace, k_dir = jax.random.split(key)
    replace = jax.random.uniform(k_replace, (_T,)) < _PROBABILITY
    direction = jnp.where(jax.random.uniform(k_dir, (_T,)) < 0.5, -1, 1)
    pos = jnp.arange(_T)
    neighbor = jnp.where(
        pos == 0, 1, jnp.where(pos == _T - 1, _T - 2, pos + direction)
    )
    return jnp.where(replace, neighbor, pos).astype(jnp.int32)


@functools.partial(
    pl.kernel,
    mesh=plsc.VectorSubcoreMesh(core_axis_name="c", subcore_axis_name="s"),
    out_type=jax.ShapeDtypeStruct((_ROWS, _T), jnp.float32),
    scratch_types=[
        pltpu.VMEM((_T,), jnp.int32),
        pltpu.VMEM((_NR, _T), jnp.float32),
        pltpu.VMEM((_NR, _T), jnp.float32),
        pltpu.VMEM((_NR, _T), jnp.float32),
        pltpu.VMEM((_NR, _T), jnp.float32),
        pltpu.SemaphoreType.DMA,
        pltpu.SemaphoreType.DMA,
        pltpu.SemaphoreType.DMA,
        pltpu.SemaphoreType.DMA,
    ],
    compiler_params=pltpu.CompilerParams(needs_layout_passes=False),
)
def _sc_jitter(x_hbm, src_hbm, out_hbm, src_v,
               in0, in1, out0, out1, sem_i0, sem_i1, sem_o0, sem_o1):
    wid = lax.axis_index("s") * 2 + lax.axis_index("c")
    row0 = wid * _RPW
    pltpu.sync_copy(src_hbm, src_v)

    ins = (in0, in1)
    outs = (out0, out1)
    sems_i = (sem_i0, sem_i1)
    sems_o = (sem_o0, sem_o1)

    def in_slice(g):
        return x_hbm.at[pl.ds(row0 + g * _NR, _NR), :]

    def out_slice(g):
        return out_hbm.at[pl.ds(row0 + g * _NR, _NR), :]

    pltpu.make_async_copy(in_slice(0), ins[0], sems_i[0]).start()

    def process(g, ph):
        pltpu.make_async_copy(in_slice(g), ins[ph], sems_i[ph]).wait()
        @pl.when(g + 1 < _NBLK)
        def _():
            pltpu.make_async_copy(
                in_slice(g + 1), ins[1 - ph], sems_i[1 - ph]
            ).start()

        @pl.when(g >= 2)
        def _():
            pltpu.make_async_copy(outs[ph], out_slice(g - 2), sems_o[ph]).wait()

        @plsc.parallel_loop(0, _T // _L, unroll=8)
        def chunk(j):
            off = j * _L
            col = src_v[pl.ds(off, _L)]
            for r in range(_NR):
                rvec = jnp.full((_L,), r, dtype=jnp.int32)
                gat = plsc.load_gather(ins[ph], [rvec, col])
                outs[ph][r, pl.ds(off, _L)] = gat

        pltpu.make_async_copy(outs[ph], out_slice(g), sems_o[ph]).start()

    def pair(i, carry):
        process(i * 2, 0)
        process(i * 2 + 1, 1)
        return carry

    lax.fori_loop(0, _NBLK // 2, pair, 0)
    pltpu.make_async_copy(outs[0], out_slice(_NBLK - 2), sems_o[0]).wait()
    pltpu.make_async_copy(outs[1], out_slice(_NBLK - 1), sems_o[1]).wait()


def kernel(quantized):
    B, C, T = quantized.shape
    x2d = quantized.reshape(B * C, T)
    src = _src_indices()
    out = _sc_jitter(x2d, src)
    return out.reshape(B, C, T)
